# baseline (speedup 1.0000x reference)
import jax
import jax.numpy as jnp
from jax.experimental import pallas as pl
from jax.experimental.pallas import tpu as pltpu


def _lookup_kernel(idx_ref, rot_blk, trans_blk, theta_ref, rho_ref):
    r = jax.lax.rem(idx_ref[0], 8)
    rows = jax.lax.broadcasted_iota(jnp.int32, (8, 3), 0)
    mask = rows == r
    theta_ref[...] = jnp.sum(jnp.where(mask, rot_blk[...], 0.0), axis=0,
                             keepdims=True)
    rho_ref[...] = jnp.sum(jnp.where(mask, trans_blk[...], 0.0), axis=0,
                           keepdims=True)


def kernel(view_ids, rot_weight, trans_weight):
    idx = view_ids[:1].astype(jnp.int32)
    grid_spec = pltpu.PrefetchScalarGridSpec(
        num_scalar_prefetch=1,
        grid=(1,),
        in_specs=[
            pl.BlockSpec((8, 3), lambda g, idx_ref: (idx_ref[0] // 8, 0)),
            pl.BlockSpec((8, 3), lambda g, idx_ref: (idx_ref[0] // 8, 0)),
        ],
        out_specs=[
            pl.BlockSpec((1, 3), lambda g, idx_ref: (0, 0)),
            pl.BlockSpec((1, 3), lambda g, idx_ref: (0, 0)),
        ],
    )
    theta, rho = pl.pallas_call(
        _lookup_kernel,
        grid_spec=grid_spec,
        out_shape=[
            jax.ShapeDtypeStruct((1, 3), jnp.float32),
            jax.ShapeDtypeStruct((1, 3), jnp.float32),
        ],
    )(idx, rot_weight, trans_weight)
    return (theta, rho)


# speedup vs baseline: 1.0219x; 1.0219x over previous
import jax
import jax.numpy as jnp
from jax.experimental import pallas as pl
from jax.experimental.pallas import tpu as pltpu


def _lookup_kernel(idx_ref, rot_ref, trans_ref, theta_ref, rho_ref,
                   theta_v, rho_v, sem):
    i = idx_ref[0]
    a = pltpu.make_async_copy(rot_ref.at[pl.ds(i, 1)], theta_v, sem)
    b = pltpu.make_async_copy(trans_ref.at[pl.ds(i, 1)], rho_v, sem)
    a.start()
    b.start()
    a.wait()
    b.wait()
    theta_ref[...] = theta_v[...]
    rho_ref[...] = rho_v[...]


def kernel(view_ids, rot_weight, trans_weight):
    idx = view_ids[:1].astype(jnp.int32)
    theta, rho = pl.pallas_call(
        _lookup_kernel,
        in_specs=[
            pl.BlockSpec(memory_space=pltpu.SMEM),
            pl.BlockSpec(memory_space=pl.ANY),
            pl.BlockSpec(memory_space=pl.ANY),
        ],
        out_specs=[
            pl.BlockSpec(memory_space=pltpu.VMEM),
            pl.BlockSpec(memory_space=pltpu.VMEM),
        ],
        out_shape=[
            jax.ShapeDtypeStruct((1, 3), jnp.float32),
            jax.ShapeDtypeStruct((1, 3), jnp.float32),
        ],
        scratch_shapes=[
            pltpu.VMEM((1, 3), jnp.float32),
            pltpu.VMEM((1, 3), jnp.float32),
            pltpu.SemaphoreType.DMA,
        ],
    )(idx, rot_weight, trans_weight)
    return (theta, rho)
